# baseline (speedup 1.0000x reference)
import functools

import jax
import jax.numpy as jnp
from jax import lax
from jax.experimental import pallas as pl
from jax.experimental.pallas import tpu as pltpu
from jax.experimental.pallas import tpu_sc as plsc

N = 10000
E = 320000
D = 64
NCONV = 3

NC = 2
NS = 16
NW = NC * NS

UNIT = 128
EPAD = 327680
UPW = EPAD // (NW * UNIT)
NBUF = 4
TPW = UPW // NBUF

EB = 8192

_mesh = plsc.VectorSubcoreMesh(core_axis_name="c", subcore_axis_name="s")
_sc_params = pltpu.CompilerParams(use_tc_tiling_on_sc=False)


@functools.partial(
    pl.kernel,
    out_type=jax.ShapeDtypeStruct((EPAD, 4, 16), jnp.float32),
    mesh=_mesh,
    scratch_types=[
        pltpu.VMEM((NBUF, UNIT), jnp.int32),
        pltpu.VMEM((NBUF, UNIT), jnp.int32),
        pltpu.VMEM((NBUF, UNIT, 4, 16), jnp.float32),
        pltpu.VMEM((NBUF, UNIT, 4, 16), jnp.float32),
        pltpu.SemaphoreType.DMA((NBUF,)),
        pltpu.SemaphoreType.DMA((NBUF,)),
        pltpu.SemaphoreType.DMA((NBUF,)),
    ],
    compiler_params=_sc_params,
)
def _sc_gather(a_hbm, b_hbm, dst_hbm, src_hbm, g_hbm,
               idx_d, idx_s, buf_a, buf_b, sem_i, sem_g, sem_o):
    wid = lax.axis_index("s") * NC + lax.axis_index("c")
    ubase = wid * UPW

    def fire_idx(u, b):
        pltpu.async_copy(dst_hbm.at[u], idx_d.at[b], sem_i.at[b])
        pltpu.async_copy(src_hbm.at[u], idx_s.at[b], sem_i.at[b])

    def drain_idx(b):
        pltpu.make_async_copy(dst_hbm.at[0], idx_d.at[b], sem_i.at[b]).wait()
        pltpu.make_async_copy(src_hbm.at[0], idx_s.at[b], sem_i.at[b]).wait()

    def drain_out(b):
        pltpu.make_async_copy(g_hbm.at[pl.ds(0, UNIT)], buf_a.at[b],
                              sem_o.at[b]).wait()

    for b in range(NBUF):
        fire_idx(ubase + b, b)

    def outer(t, _):
        cps = []
        for b in range(NBUF):
            @pl.when(t >= 1)
            def _():
                drain_out(b)
            drain_idx(b)
            cps.append(pltpu.async_copy(
                a_hbm.at[idx_d.at[b]], buf_a.at[b], sem_g.at[b]))
            cps.append(pltpu.async_copy(
                b_hbm.at[idx_s.at[b]], buf_b.at[b], sem_g.at[b]))
        for b in range(NBUF):
            cps[2 * b].wait()
            cps[2 * b + 1].wait()

            @pl.when(t < TPW - 1)
            def _():
                fire_idx(ubase + (t + 1) * NBUF + b, b)

            def addrow(r, _):
                for rr in range(4):
                    for k in range(4):
                        buf_a[b, 4 * r + rr, k] = (buf_a[b, 4 * r + rr, k]
                                                   + buf_b[b, 4 * r + rr, k])
                return 0

            lax.fori_loop(0, UNIT // 4, addrow, 0)
            off = (ubase + t * NBUF + b) * UNIT
            pltpu.async_copy(buf_a.at[b], g_hbm.at[pl.ds(off, UNIT)],
                             sem_o.at[b])
        return 0

    lax.fori_loop(0, TPW, outer, 0)
    for b in range(NBUF):
        drain_out(b)


@functools.partial(
    pl.kernel,
    out_type=jax.ShapeDtypeStruct((NC, N, 4, 16), jnp.float32),
    mesh=_mesh,
    scratch_types=[
        pltpu.VMEM((NBUF, UNIT), jnp.int32),
        pltpu.VMEM((NBUF, UNIT, 4, 16), jnp.float32),
        pltpu.VMEM_SHARED((N, 4, 16), jnp.float32),
        pltpu.SemaphoreType.DMA((NBUF,)),
        pltpu.SemaphoreType.DMA((NBUF,)),
    ],
    compiler_params=_sc_params,
)
def _sc_scatter(m2_hbm, dst_hbm, zeros_hbm, out_hbm, idx_d, buf, aggr_sh,
                sem_l, sem_s):
    cid = lax.axis_index("c")
    sid = lax.axis_index("s")
    wid = sid * NC + cid
    ubase = wid * UPW

    row0 = sid * 624
    pltpu.sync_copy(zeros_hbm.at[pl.ds(row0, 624)], aggr_sh.at[pl.ds(row0, 624)])

    @pl.when(sid == NS - 1)
    def _():
        pltpu.sync_copy(zeros_hbm.at[pl.ds(9984, 16)], aggr_sh.at[pl.ds(9984, 16)])

    plsc.subcore_barrier()

    def fire_load(u, b):
        pltpu.async_copy(dst_hbm.at[u], idx_d.at[b], sem_l.at[b])
        pltpu.async_copy(m2_hbm.at[pl.ds(u * UNIT, UNIT)], buf.at[b],
                         sem_l.at[b])

    def drain_load(b):
        pltpu.make_async_copy(dst_hbm.at[0], idx_d.at[b], sem_l.at[b]).wait()
        pltpu.make_async_copy(m2_hbm.at[pl.ds(0, UNIT)], buf.at[b],
                              sem_l.at[b]).wait()

    for b in range(NBUF):
        fire_load(ubase + b, b)

    def outer(t, _):
        cps = []
        for b in range(NBUF):
            drain_load(b)
            cps.append(pltpu.async_copy(
                buf.at[b], aggr_sh.at[idx_d.at[b]], sem_s.at[b], add=True))
        for b in range(NBUF):
            cps[b].wait()

            @pl.when(t < TPW - 1)
            def _():
                fire_load(ubase + (t + 1) * NBUF + b, b)
        return 0

    lax.fori_loop(0, TPW, outer, 0)
    plsc.subcore_barrier()

    pltpu.sync_copy(aggr_sh.at[pl.ds(row0, 624)],
                    out_hbm.at[cid, pl.ds(row0, 624)])

    @pl.when(sid == NS - 1)
    def _():
        pltpu.sync_copy(aggr_sh.at[pl.ds(9984, 16)],
                        out_hbm.at[cid, pl.ds(9984, 16)])


def _edge_body(ea_ref, g_ref, epw_ref, epb_ref, w1c_ref, w2_ref,
               b2_ref, o_ref):
    i = pl.program_id(0)
    ea = ea_ref[...]
    e = jax.nn.softplus(ea[:, 0:1] * epw_ref[0:1, :]
                        + ea[:, 1:2] * epw_ref[1:2, :] + epb_ref[...])
    g = g_ref[...] + jnp.dot(
        e, w1c_ref[...], preferred_element_type=jnp.float32)
    m = jax.nn.softplus(g)
    m2 = jax.nn.softplus(
        jnp.dot(m, w2_ref[...], preferred_element_type=jnp.float32)
        + b2_ref[...])
    rows = i * EB + lax.broadcasted_iota(jnp.int32, (EB, 1), 0)
    o_ref[...] = jnp.where(rows < E, m2, 0.0)


def _edge_mlp(eap, g, epw, epb, w1c, w2, b2):
    return pl.pallas_call(
        _edge_body,
        grid=(EPAD // EB,),
        in_specs=[
            pl.BlockSpec((EB, 2), lambda i: (i, 0)),
            pl.BlockSpec((EB, D), lambda i: (i, 0)),
            pl.BlockSpec((2, 32), lambda i: (0, 0)),
            pl.BlockSpec((1, 32), lambda i: (0, 0)),
            pl.BlockSpec((32, D), lambda i: (0, 0)),
            pl.BlockSpec((D, D), lambda i: (0, 0)),
            pl.BlockSpec((1, D), lambda i: (0, 0)),
        ],
        out_specs=pl.BlockSpec((EB, D), lambda i: (i, 0)),
        out_shape=jax.ShapeDtypeStruct((EPAD, D), jnp.float32),
    )(eap, g, epw, epb, w1c, w2, b2)


def _bn(t, gm, bt):
    mean = jnp.mean(t, axis=0, keepdims=True)
    var = jnp.mean((t - mean) ** 2, axis=0, keepdims=True)
    return gm * (t - mean) / jnp.sqrt(var + 1e-5) + bt


def _init_body(xp_ref, npw_ref, npb_ref, gm_ref, bt_ref, w1a_ref, b1_ref,
               w1b_ref, h_ref, a_ref, b_ref):
    t = jax.nn.softplus(
        jnp.dot(xp_ref[...], npw_ref[...], preferred_element_type=jnp.float32)
        + npb_ref[...])
    h = _bn(t, gm_ref[...], bt_ref[...])
    h_ref[...] = h
    a_ref[...] = jnp.dot(h, w1a_ref[...],
                         preferred_element_type=jnp.float32) + b1_ref[...]
    b_ref[...] = jnp.dot(h, w1b_ref[...], preferred_element_type=jnp.float32)


def _node_init(xp, npw, npb, gm, bt, w1a, b1, w1b):
    return pl.pallas_call(
        _init_body,
        out_shape=[jax.ShapeDtypeStruct((N, D), jnp.float32)] * 3,
    )(xp, npw, npb, gm, bt, w1a, b1, w1b)


def _update_body(h_ref, p0_ref, p1_ref, w3a_ref, w3b_ref, b3_ref, w4_ref,
                 b4_ref, gm_ref, bt_ref, w1a_ref, b1_ref, w1b_ref,
                 h_ref_o, a_ref_o, b_ref_o):
    h = h_ref[...]
    aggr = p0_ref[...] + p1_ref[...]
    upd = jax.nn.softplus(
        jnp.dot(h, w3a_ref[...], preferred_element_type=jnp.float32)
        + jnp.dot(aggr, w3b_ref[...], preferred_element_type=jnp.float32)
        + b3_ref[...])
    t = jnp.dot(upd, w4_ref[...],
                preferred_element_type=jnp.float32) + b4_ref[...] + h
    hn = _bn(t, gm_ref[...], bt_ref[...])
    h_ref_o[...] = hn
    a_ref_o[...] = jnp.dot(hn, w1a_ref[...],
                           preferred_element_type=jnp.float32) + b1_ref[...]
    b_ref_o[...] = jnp.dot(hn, w1b_ref[...], preferred_element_type=jnp.float32)


def _node_update(h, p0, p1, w3a, w3b, b3, w4, b4, gm, bt, w1a, b1, w1b):
    return pl.pallas_call(
        _update_body,
        out_shape=[jax.ShapeDtypeStruct((N, D), jnp.float32)] * 3,
    )(h, p0, p1, w3a, w3b, b3, w4, b4, gm, bt, w1a, b1, w1b)


def _final_body(h_ref, p0_ref, p1_ref, w3a_ref, w3b_ref, b3_ref, w4_ref,
                b4_ref, gm_ref, bt_ref, ow1_ref, ob1_ref, ow2_ref, ob2_ref,
                o_ref):
    h = h_ref[...]
    aggr = p0_ref[...] + p1_ref[...]
    upd = jax.nn.softplus(
        jnp.dot(h, w3a_ref[...], preferred_element_type=jnp.float32)
        + jnp.dot(aggr, w3b_ref[...], preferred_element_type=jnp.float32)
        + b3_ref[...])
    t = jnp.dot(upd, w4_ref[...],
                preferred_element_type=jnp.float32) + b4_ref[...] + h
    hn = _bn(t, gm_ref[...], bt_ref[...])
    pooled = jnp.mean(hn, axis=0, keepdims=True)
    o1 = jax.nn.softplus(
        jnp.dot(pooled, ow1_ref[...], preferred_element_type=jnp.float32)
        + ob1_ref[...])
    o_ref[...] = jnp.dot(o1, ow2_ref[...],
                         preferred_element_type=jnp.float32) + ob2_ref[...]


def _node_final(h, p0, p1, w3a, w3b, b3, w4, b4, gm, bt, ow1, ob1, ow2, ob2):
    return pl.pallas_call(
        _final_body,
        out_shape=jax.ShapeDtypeStruct((1, 1), jnp.float32),
    )(h, p0, p1, w3a, w3b, b3, w4, b4, gm, bt, ow1, ob1, ow2, ob2)


def kernel(x, edge_attr, params, edge_index, batch):
    del batch
    src = edge_index[0]
    dst = edge_index[1]
    pad_e = EPAD - E
    dstp = jnp.concatenate([dst, jnp.zeros((pad_e,), jnp.int32)])
    srcp = jnp.concatenate([src, jnp.zeros((pad_e,), jnp.int32)])
    dst2 = dstp.reshape(EPAD // UNIT, UNIT)
    src2 = srcp.reshape(EPAD // UNIT, UNIT)
    zeros_sc = jnp.zeros((N, 4, 16), jnp.float32)
    eap = jnp.pad(edge_attr, ((0, pad_e), (0, 0)))

    xp = jnp.pad(x, ((0, 0), (0, 3)))
    npw = jnp.pad(params["np_W"], ((0, 3), (0, 0)))
    row = lambda v: v.reshape(1, -1)

    w1 = params["conv_W1"]
    w1a = [w1[l, :D] for l in range(NCONV)]
    w1b = [w1[l, D:2 * D] for l in range(NCONV)]
    w1c = [w1[l, 2 * D:] for l in range(NCONV)]
    b1 = [row(params["conv_b1"][l]) for l in range(NCONV)]
    w2 = [params["conv_W2"][l] for l in range(NCONV)]
    b2 = [row(params["conv_b2"][l]) for l in range(NCONV)]
    w3 = params["conv_W3"]
    w3a = [w3[l, :D] for l in range(NCONV)]
    w3b = [w3[l, D:] for l in range(NCONV)]
    b3 = [row(params["conv_b3"][l]) for l in range(NCONV)]
    w4 = [params["conv_W4"][l] for l in range(NCONV)]
    b4 = [row(params["conv_b4"][l]) for l in range(NCONV)]
    gm = [row(params["bn_gamma"][l]) for l in range(NCONV)]
    bt = [row(params["bn_beta"][l]) for l in range(NCONV)]

    h, a, b = _node_init(xp, npw, row(params["np_b"]),
                         row(params["np_gamma"]), row(params["np_beta"]),
                         w1a[0], b1[0], w1b[0])

    for l in range(NCONV):
        g = _sc_gather(a.reshape(N, 4, 16), b.reshape(N, 4, 16), dst2, src2)
        m2 = _edge_mlp(eap, g.reshape(EPAD, D), params["ep_W"],
                       row(params["ep_b"]), w1c[l], w2[l], b2[l])
        p = _sc_scatter(m2.reshape(EPAD, 4, 16), dst2, zeros_sc)
        p = p.reshape(NC, N, D)
        if l < NCONV - 1:
            h, a, b = _node_update(h, p[0], p[1], w3a[l], w3b[l], b3[l],
                                   w4[l], b4[l], gm[l], bt[l],
                                   w1a[l + 1], b1[l + 1], w1b[l + 1])
        else:
            o = _node_final(h, p[0], p[1], w3a[l], w3b[l], b3[l], w4[l],
                            b4[l], gm[l], bt[l], params["out_W1"],
                            row(params["out_b1"]), params["out_W2"],
                            row(params["out_b2"]))
    return o


# speedup vs baseline: 2.1477x; 2.1477x over previous
import functools

import jax
import jax.numpy as jnp
from jax import lax
from jax.experimental import pallas as pl
from jax.experimental.pallas import tpu as pltpu
from jax.experimental.pallas import tpu_sc as plsc

N = 10000
E = 320000
D = 64
NCONV = 3

NC = 2
NS = 16
NW = NC * NS

UNIT = 128
EPAD = 327680
UPW = EPAD // (NW * UNIT)
NBUF = 4
TPW = UPW // NBUF

EB = 8192

_mesh = plsc.VectorSubcoreMesh(core_axis_name="c", subcore_axis_name="s")
_sc_params = pltpu.CompilerParams(use_tc_tiling_on_sc=False)


@functools.partial(
    pl.kernel,
    out_type=jax.ShapeDtypeStruct((EPAD, D), jnp.float32),
    mesh=_mesh,
    scratch_types=[
        pltpu.VMEM((NBUF, UNIT), jnp.int32),
        pltpu.VMEM((NBUF, UNIT), jnp.int32),
        pltpu.VMEM((NBUF, UNIT, D), jnp.float32),
        pltpu.VMEM((NBUF, UNIT, D), jnp.float32),
        pltpu.SemaphoreType.DMA((NBUF,)),
        pltpu.SemaphoreType.DMA((NBUF,)),
        pltpu.SemaphoreType.DMA((NBUF,)),
    ],
    compiler_params=_sc_params,
)
def _sc_gather(a_hbm, b_hbm, dst_hbm, src_hbm, g_hbm,
               idx_d, idx_s, buf_a, buf_b, sem_i, sem_g, sem_o):
    wid = lax.axis_index("s") * NC + lax.axis_index("c")
    ubase = wid * UPW

    def fire_idx(u, b):
        pltpu.async_copy(dst_hbm.at[u], idx_d.at[b], sem_i.at[b])
        pltpu.async_copy(src_hbm.at[u], idx_s.at[b], sem_i.at[b])

    def drain_idx(b):
        pltpu.make_async_copy(dst_hbm.at[0], idx_d.at[b], sem_i.at[b]).wait()
        pltpu.make_async_copy(src_hbm.at[0], idx_s.at[b], sem_i.at[b]).wait()

    def drain_out(b):
        pltpu.make_async_copy(g_hbm.at[pl.ds(0, UNIT)], buf_a.at[b],
                              sem_o.at[b]).wait()

    for b in range(NBUF):
        fire_idx(ubase + b, b)

    def outer(t, _):
        cps = []
        for b in range(NBUF):
            @pl.when(t >= 1)
            def _():
                drain_out(b)
            drain_idx(b)
            cps.append(pltpu.async_copy(
                a_hbm.at[idx_d.at[b]], buf_a.at[b], sem_g.at[b]))
            cps.append(pltpu.async_copy(
                b_hbm.at[idx_s.at[b]], buf_b.at[b], sem_g.at[b]))
        for b in range(NBUF):
            cps[2 * b].wait()
            cps[2 * b + 1].wait()

            @pl.when(t < TPW - 1)
            def _():
                fire_idx(ubase + (t + 1) * NBUF + b, b)

            def addrow(r, _):
                for rr in range(4):
                    for k in range(4):
                        s = pl.ds(16 * k, 16)
                        buf_a[b, 4 * r + rr, s] = (buf_a[b, 4 * r + rr, s]
                                                   + buf_b[b, 4 * r + rr, s])
                return 0

            lax.fori_loop(0, UNIT // 4, addrow, 0)
            off = (ubase + t * NBUF + b) * UNIT
            pltpu.async_copy(buf_a.at[b], g_hbm.at[pl.ds(off, UNIT)],
                             sem_o.at[b])
        return 0

    lax.fori_loop(0, TPW, outer, 0)
    for b in range(NBUF):
        drain_out(b)


@functools.partial(
    pl.kernel,
    out_type=jax.ShapeDtypeStruct((NC, N, D), jnp.float32),
    mesh=_mesh,
    scratch_types=[
        pltpu.VMEM((NBUF, UNIT), jnp.int32),
        pltpu.VMEM((NBUF, UNIT, D), jnp.float32),
        pltpu.VMEM_SHARED((N, D), jnp.float32),
        pltpu.SemaphoreType.DMA((NBUF,)),
        pltpu.SemaphoreType.DMA((NBUF,)),
    ],
    compiler_params=_sc_params,
)
def _sc_scatter(m2_hbm, dst_hbm, zeros_hbm, out_hbm, idx_d, buf, aggr_sh,
                sem_l, sem_s):
    cid = lax.axis_index("c")
    sid = lax.axis_index("s")
    wid = sid * NC + cid
    ubase = wid * UPW

    row0 = sid * 624
    pltpu.sync_copy(zeros_hbm.at[pl.ds(row0, 624)], aggr_sh.at[pl.ds(row0, 624)])

    @pl.when(sid == NS - 1)
    def _():
        pltpu.sync_copy(zeros_hbm.at[pl.ds(9984, 16)], aggr_sh.at[pl.ds(9984, 16)])

    plsc.subcore_barrier()

    def fire_load(u, b):
        pltpu.async_copy(dst_hbm.at[u], idx_d.at[b], sem_l.at[b])
        pltpu.async_copy(m2_hbm.at[pl.ds(u * UNIT, UNIT)], buf.at[b],
                         sem_l.at[b])

    def drain_load(b):
        pltpu.make_async_copy(dst_hbm.at[0], idx_d.at[b], sem_l.at[b]).wait()
        pltpu.make_async_copy(m2_hbm.at[pl.ds(0, UNIT)], buf.at[b],
                              sem_l.at[b]).wait()

    for b in range(NBUF):
        fire_load(ubase + b, b)

    def outer(t, _):
        cps = []
        for b in range(NBUF):
            drain_load(b)
            cps.append(pltpu.async_copy(
                buf.at[b], aggr_sh.at[idx_d.at[b]], sem_s.at[b], add=True))
        for b in range(NBUF):
            cps[b].wait()

            @pl.when(t < TPW - 1)
            def _():
                fire_load(ubase + (t + 1) * NBUF + b, b)
        return 0

    lax.fori_loop(0, TPW, outer, 0)
    plsc.subcore_barrier()

    pltpu.sync_copy(aggr_sh.at[pl.ds(row0, 624)],
                    out_hbm.at[cid, pl.ds(row0, 624)])

    @pl.when(sid == NS - 1)
    def _():
        pltpu.sync_copy(aggr_sh.at[pl.ds(9984, 16)],
                        out_hbm.at[cid, pl.ds(9984, 16)])


def _edge_body(ea_ref, g_ref, epw_ref, epb_ref, w1c_ref, w2_ref,
               b2_ref, o_ref):
    i = pl.program_id(0)
    ea = ea_ref[...]
    e = jax.nn.softplus(ea[:, 0:1] * epw_ref[0:1, :]
                        + ea[:, 1:2] * epw_ref[1:2, :] + epb_ref[...])
    g = g_ref[...] + jnp.dot(
        e, w1c_ref[...], preferred_element_type=jnp.float32)
    m = jax.nn.softplus(g)
    m2 = jax.nn.softplus(
        jnp.dot(m, w2_ref[...], preferred_element_type=jnp.float32)
        + b2_ref[...])
    rows = i * EB + lax.broadcasted_iota(jnp.int32, (EB, 1), 0)
    o_ref[...] = jnp.where(rows < E, m2, 0.0)


def _edge_mlp(eap, g, epw, epb, w1c, w2, b2):
    return pl.pallas_call(
        _edge_body,
        grid=(EPAD // EB,),
        in_specs=[
            pl.BlockSpec((EB, 2), lambda i: (i, 0)),
            pl.BlockSpec((EB, D), lambda i: (i, 0)),
            pl.BlockSpec((2, 32), lambda i: (0, 0)),
            pl.BlockSpec((1, 32), lambda i: (0, 0)),
            pl.BlockSpec((32, D), lambda i: (0, 0)),
            pl.BlockSpec((D, D), lambda i: (0, 0)),
            pl.BlockSpec((1, D), lambda i: (0, 0)),
        ],
        out_specs=pl.BlockSpec((EB, D), lambda i: (i, 0)),
        out_shape=jax.ShapeDtypeStruct((EPAD, D), jnp.float32),
    )(eap, g, epw, epb, w1c, w2, b2)


def _bn(t, gm, bt):
    mean = jnp.mean(t, axis=0, keepdims=True)
    var = jnp.mean((t - mean) ** 2, axis=0, keepdims=True)
    return gm * (t - mean) / jnp.sqrt(var + 1e-5) + bt


def _init_body(xp_ref, npw_ref, npb_ref, gm_ref, bt_ref, w1a_ref, b1_ref,
               w1b_ref, h_ref, a_ref, b_ref):
    t = jax.nn.softplus(
        jnp.dot(xp_ref[...], npw_ref[...], preferred_element_type=jnp.float32)
        + npb_ref[...])
    h = _bn(t, gm_ref[...], bt_ref[...])
    h_ref[...] = h
    a_ref[...] = jnp.dot(h, w1a_ref[...],
                         preferred_element_type=jnp.float32) + b1_ref[...]
    b_ref[...] = jnp.dot(h, w1b_ref[...], preferred_element_type=jnp.float32)


def _node_init(xp, npw, npb, gm, bt, w1a, b1, w1b):
    return pl.pallas_call(
        _init_body,
        out_shape=[jax.ShapeDtypeStruct((N, D), jnp.float32)] * 3,
    )(xp, npw, npb, gm, bt, w1a, b1, w1b)


def _update_body(h_ref, p0_ref, p1_ref, w3a_ref, w3b_ref, b3_ref, w4_ref,
                 b4_ref, gm_ref, bt_ref, w1a_ref, b1_ref, w1b_ref,
                 h_ref_o, a_ref_o, b_ref_o):
    h = h_ref[...]
    aggr = p0_ref[...] + p1_ref[...]
    upd = jax.nn.softplus(
        jnp.dot(h, w3a_ref[...], preferred_element_type=jnp.float32)
        + jnp.dot(aggr, w3b_ref[...], preferred_element_type=jnp.float32)
        + b3_ref[...])
    t = jnp.dot(upd, w4_ref[...],
                preferred_element_type=jnp.float32) + b4_ref[...] + h
    hn = _bn(t, gm_ref[...], bt_ref[...])
    h_ref_o[...] = hn
    a_ref_o[...] = jnp.dot(hn, w1a_ref[...],
                           preferred_element_type=jnp.float32) + b1_ref[...]
    b_ref_o[...] = jnp.dot(hn, w1b_ref[...], preferred_element_type=jnp.float32)


def _node_update(h, p0, p1, w3a, w3b, b3, w4, b4, gm, bt, w1a, b1, w1b):
    return pl.pallas_call(
        _update_body,
        out_shape=[jax.ShapeDtypeStruct((N, D), jnp.float32)] * 3,
    )(h, p0, p1, w3a, w3b, b3, w4, b4, gm, bt, w1a, b1, w1b)


def _final_body(h_ref, p0_ref, p1_ref, w3a_ref, w3b_ref, b3_ref, w4_ref,
                b4_ref, gm_ref, bt_ref, ow1_ref, ob1_ref, ow2_ref, ob2_ref,
                o_ref):
    h = h_ref[...]
    aggr = p0_ref[...] + p1_ref[...]
    upd = jax.nn.softplus(
        jnp.dot(h, w3a_ref[...], preferred_element_type=jnp.float32)
        + jnp.dot(aggr, w3b_ref[...], preferred_element_type=jnp.float32)
        + b3_ref[...])
    t = jnp.dot(upd, w4_ref[...],
                preferred_element_type=jnp.float32) + b4_ref[...] + h
    hn = _bn(t, gm_ref[...], bt_ref[...])
    pooled = jnp.mean(hn, axis=0, keepdims=True)
    o1 = jax.nn.softplus(
        jnp.dot(pooled, ow1_ref[...], preferred_element_type=jnp.float32)
        + ob1_ref[...])
    o_ref[...] = jnp.dot(o1, ow2_ref[...],
                         preferred_element_type=jnp.float32) + ob2_ref[...]


def _node_final(h, p0, p1, w3a, w3b, b3, w4, b4, gm, bt, ow1, ob1, ow2, ob2):
    return pl.pallas_call(
        _final_body,
        out_shape=jax.ShapeDtypeStruct((1, 1), jnp.float32),
    )(h, p0, p1, w3a, w3b, b3, w4, b4, gm, bt, ow1, ob1, ow2, ob2)


def kernel(x, edge_attr, params, edge_index, batch):
    del batch
    src = edge_index[0]
    dst = edge_index[1]
    pad_e = EPAD - E
    dstp = jnp.concatenate([dst, jnp.zeros((pad_e,), jnp.int32)])
    srcp = jnp.concatenate([src, jnp.zeros((pad_e,), jnp.int32)])
    dst2 = dstp.reshape(EPAD // UNIT, UNIT)
    src2 = srcp.reshape(EPAD // UNIT, UNIT)
    zeros_sc = jnp.zeros((N, D), jnp.float32)
    eap = jnp.pad(edge_attr, ((0, pad_e), (0, 0)))

    xp = jnp.pad(x, ((0, 0), (0, 3)))
    npw = jnp.pad(params["np_W"], ((0, 3), (0, 0)))
    row = lambda v: v.reshape(1, -1)

    w1 = params["conv_W1"]
    w1a = [w1[l, :D] for l in range(NCONV)]
    w1b = [w1[l, D:2 * D] for l in range(NCONV)]
    w1c = [w1[l, 2 * D:] for l in range(NCONV)]
    b1 = [row(params["conv_b1"][l]) for l in range(NCONV)]
    w2 = [params["conv_W2"][l] for l in range(NCONV)]
    b2 = [row(params["conv_b2"][l]) for l in range(NCONV)]
    w3 = params["conv_W3"]
    w3a = [w3[l, :D] for l in range(NCONV)]
    w3b = [w3[l, D:] for l in range(NCONV)]
    b3 = [row(params["conv_b3"][l]) for l in range(NCONV)]
    w4 = [params["conv_W4"][l] for l in range(NCONV)]
    b4 = [row(params["conv_b4"][l]) for l in range(NCONV)]
    gm = [row(params["bn_gamma"][l]) for l in range(NCONV)]
    bt = [row(params["bn_beta"][l]) for l in range(NCONV)]

    h, a, b = _node_init(xp, npw, row(params["np_b"]),
                         row(params["np_gamma"]), row(params["np_beta"]),
                         w1a[0], b1[0], w1b[0])

    for l in range(NCONV):
        g = _sc_gather(a, b, dst2, src2)
        m2 = _edge_mlp(eap, g, params["ep_W"],
                       row(params["ep_b"]), w1c[l], w2[l], b2[l])
        p = _sc_scatter(m2, dst2, zeros_sc)
        if l < NCONV - 1:
            h, a, b = _node_update(h, p[0], p[1], w3a[l], w3b[l], b3[l],
                                   w4[l], b4[l], gm[l], bt[l],
                                   w1a[l + 1], b1[l + 1], w1b[l + 1])
        else:
            o = _node_final(h, p[0], p[1], w3a[l], w3b[l], b3[l], w4[l],
                            b4[l], gm[l], bt[l], params["out_W1"],
                            row(params["out_b1"]), params["out_W2"],
                            row(params["out_b2"]))
    return o


# speedup vs baseline: 2.5663x; 1.1949x over previous
import functools

import jax
import jax.numpy as jnp
from jax import lax
from jax.experimental import pallas as pl
from jax.experimental.pallas import tpu as pltpu
from jax.experimental.pallas import tpu_sc as plsc

N = 10000
E = 320000
D = 64
NCONV = 3

NC = 2
NS = 16
NW = NC * NS

UNIT = 128
EPAD = 327680
NCHUNK = 2
CU = EPAD // (NCHUNK * UNIT)
CE = CU * UNIT
UPW = CU // NW
NBUF = 4
TPW = UPW // NBUF

EB = 8192

_mesh = plsc.VectorSubcoreMesh(core_axis_name="c", subcore_axis_name="s")
_sc_params = pltpu.CompilerParams(use_tc_tiling_on_sc=False)


@functools.partial(
    pl.kernel,
    out_type=jax.ShapeDtypeStruct((CE, D), jnp.float32),
    mesh=_mesh,
    scratch_types=[
        pltpu.VMEM((NBUF, UNIT), jnp.int32),
        pltpu.VMEM((NBUF, UNIT), jnp.int32),
        pltpu.VMEM((NBUF, UNIT, D), jnp.float32),
        pltpu.VMEM((NBUF, UNIT, D), jnp.float32),
        pltpu.SemaphoreType.DMA((NBUF,)),
        pltpu.SemaphoreType.DMA((NBUF,)),
        pltpu.SemaphoreType.DMA((NBUF,)),
    ],
    compiler_params=_sc_params,
)
def _sc_gather(a_hbm, b_hbm, dst_hbm, src_hbm, g_hbm,
               idx_d, idx_s, buf_a, buf_b, sem_i, sem_g, sem_o):
    wid = lax.axis_index("s") * NC + lax.axis_index("c")
    ubase = wid * UPW

    def fire_idx(u, b):
        pltpu.async_copy(dst_hbm.at[u], idx_d.at[b], sem_i.at[b])
        pltpu.async_copy(src_hbm.at[u], idx_s.at[b], sem_i.at[b])

    def drain_idx(b):
        pltpu.make_async_copy(dst_hbm.at[0], idx_d.at[b], sem_i.at[b]).wait()
        pltpu.make_async_copy(src_hbm.at[0], idx_s.at[b], sem_i.at[b]).wait()

    def drain_out(b):
        pltpu.make_async_copy(g_hbm.at[pl.ds(0, UNIT)], buf_a.at[b],
                              sem_o.at[b]).wait()

    for b in range(NBUF):
        fire_idx(ubase + b, b)

    def outer(t, _):
        cps = []
        for b in range(NBUF):
            @pl.when(t >= 1)
            def _():
                drain_out(b)
            drain_idx(b)
            cps.append(pltpu.async_copy(
                a_hbm.at[idx_d.at[b]], buf_a.at[b], sem_g.at[b]))
            cps.append(pltpu.async_copy(
                b_hbm.at[idx_s.at[b]], buf_b.at[b], sem_g.at[b]))
        for b in range(NBUF):
            cps[2 * b].wait()
            cps[2 * b + 1].wait()

            @pl.when(t < TPW - 1)
            def _():
                fire_idx(ubase + (t + 1) * NBUF + b, b)

            def addrow(r, _):
                for rr in range(4):
                    for k in range(4):
                        s = pl.ds(16 * k, 16)
                        buf_a[b, 4 * r + rr, s] = (buf_a[b, 4 * r + rr, s]
                                                   + buf_b[b, 4 * r + rr, s])
                return 0

            lax.fori_loop(0, UNIT // 4, addrow, 0)
            off = (ubase + t * NBUF + b) * UNIT
            pltpu.async_copy(buf_a.at[b], g_hbm.at[pl.ds(off, UNIT)],
                             sem_o.at[b])
        return 0

    lax.fori_loop(0, TPW, outer, 0)
    for b in range(NBUF):
        drain_out(b)


@functools.partial(
    pl.kernel,
    out_type=jax.ShapeDtypeStruct((NC, N, D), jnp.float32),
    mesh=_mesh,
    scratch_types=[
        pltpu.VMEM((NBUF, UNIT), jnp.int32),
        pltpu.VMEM((NBUF, UNIT, D), jnp.float32),
        pltpu.VMEM_SHARED((N, D), jnp.float32),
        pltpu.SemaphoreType.DMA((NBUF,)),
        pltpu.SemaphoreType.DMA((NBUF,)),
    ],
    compiler_params=_sc_params,
)
def _sc_scatter(m2_hbm, dst_hbm, zeros_hbm, out_hbm, idx_d, buf, aggr_sh,
                sem_l, sem_s):
    cid = lax.axis_index("c")
    sid = lax.axis_index("s")
    wid = sid * NC + cid
    ubase = wid * UPW

    row0 = sid * 624
    pltpu.sync_copy(zeros_hbm.at[pl.ds(row0, 624)], aggr_sh.at[pl.ds(row0, 624)])

    @pl.when(sid == NS - 1)
    def _():
        pltpu.sync_copy(zeros_hbm.at[pl.ds(9984, 16)], aggr_sh.at[pl.ds(9984, 16)])

    plsc.subcore_barrier()

    def fire_load(u, b):
        pltpu.async_copy(dst_hbm.at[u], idx_d.at[b], sem_l.at[b])
        pltpu.async_copy(m2_hbm.at[pl.ds(u * UNIT, UNIT)], buf.at[b],
                         sem_l.at[b])

    def drain_load(b):
        pltpu.make_async_copy(dst_hbm.at[0], idx_d.at[b], sem_l.at[b]).wait()
        pltpu.make_async_copy(m2_hbm.at[pl.ds(0, UNIT)], buf.at[b],
                              sem_l.at[b]).wait()

    for b in range(NBUF):
        fire_load(ubase + b, b)

    def outer(t, _):
        cps = []
        for b in range(NBUF):
            drain_load(b)
            cps.append(pltpu.async_copy(
                buf.at[b], aggr_sh.at[idx_d.at[b]], sem_s.at[b], add=True))
        for b in range(NBUF):
            cps[b].wait()

            @pl.when(t < TPW - 1)
            def _():
                fire_load(ubase + (t + 1) * NBUF + b, b)
        return 0

    lax.fori_loop(0, TPW, outer, 0)
    plsc.subcore_barrier()

    pltpu.sync_copy(aggr_sh.at[pl.ds(row0, 624)],
                    out_hbm.at[cid, pl.ds(row0, 624)])

    @pl.when(sid == NS - 1)
    def _():
        pltpu.sync_copy(aggr_sh.at[pl.ds(9984, 16)],
                        out_hbm.at[cid, pl.ds(9984, 16)])


def _make_edge_body(row0):
    def _edge_body(ea_ref, g_ref, epw_ref, epb_ref, w1c_ref, w2_ref,
                   b2_ref, o_ref):
        i = pl.program_id(0)
        ea = ea_ref[...]
        e = jax.nn.softplus(ea[:, 0:1] * epw_ref[0:1, :]
                            + ea[:, 1:2] * epw_ref[1:2, :] + epb_ref[...])
        g = g_ref[...] + jnp.dot(
            e, w1c_ref[...], preferred_element_type=jnp.float32)
        m = jax.nn.softplus(g)
        m2 = jax.nn.softplus(
            jnp.dot(m, w2_ref[...], preferred_element_type=jnp.float32)
            + b2_ref[...])
        if row0 + CE > E:
            rows = row0 + i * EB + lax.broadcasted_iota(jnp.int32, (EB, 1), 0)
            m2 = jnp.where(rows < E, m2, 0.0)
        o_ref[...] = m2
    return _edge_body


_edge_bodies = [_make_edge_body(c * CE) for c in range(NCHUNK)]


def _edge_mlp(c, eap_c, g, epw, epb, w1c, w2, b2):
    return pl.pallas_call(
        _edge_bodies[c],
        grid=(CE // EB,),
        in_specs=[
            pl.BlockSpec((EB, 2), lambda i: (i, 0)),
            pl.BlockSpec((EB, D), lambda i: (i, 0)),
            pl.BlockSpec((2, 32), lambda i: (0, 0)),
            pl.BlockSpec((1, 32), lambda i: (0, 0)),
            pl.BlockSpec((32, D), lambda i: (0, 0)),
            pl.BlockSpec((D, D), lambda i: (0, 0)),
            pl.BlockSpec((1, D), lambda i: (0, 0)),
        ],
        out_specs=pl.BlockSpec((EB, D), lambda i: (i, 0)),
        out_shape=jax.ShapeDtypeStruct((CE, D), jnp.float32),
    )(eap_c, g, epw, epb, w1c, w2, b2)


def _bn(t, gm, bt):
    mean = jnp.mean(t, axis=0, keepdims=True)
    var = jnp.mean((t - mean) ** 2, axis=0, keepdims=True)
    return gm * (t - mean) / jnp.sqrt(var + 1e-5) + bt


def _init_body(xp_ref, npw_ref, npb_ref, gm_ref, bt_ref, w1a_ref, b1_ref,
               w1b_ref, h_ref, a_ref, b_ref):
    t = jax.nn.softplus(
        jnp.dot(xp_ref[...], npw_ref[...], preferred_element_type=jnp.float32)
        + npb_ref[...])
    h = _bn(t, gm_ref[...], bt_ref[...])
    h_ref[...] = h
    a_ref[...] = jnp.dot(h, w1a_ref[...],
                         preferred_element_type=jnp.float32) + b1_ref[...]
    b_ref[...] = jnp.dot(h, w1b_ref[...], preferred_element_type=jnp.float32)


def _node_init(xp, npw, npb, gm, bt, w1a, b1, w1b):
    return pl.pallas_call(
        _init_body,
        out_shape=[jax.ShapeDtypeStruct((N, D), jnp.float32)] * 3,
    )(xp, npw, npb, gm, bt, w1a, b1, w1b)


def _update_body(h_ref, p0_ref, p1_ref, w3a_ref, w3b_ref, b3_ref, w4_ref,
                 b4_ref, gm_ref, bt_ref, w1a_ref, b1_ref, w1b_ref,
                 h_ref_o, a_ref_o, b_ref_o):
    h = h_ref[...]
    aggr = (p0_ref[0] + p0_ref[1]) + (p1_ref[0] + p1_ref[1])
    upd = jax.nn.softplus(
        jnp.dot(h, w3a_ref[...], preferred_element_type=jnp.float32)
        + jnp.dot(aggr, w3b_ref[...], preferred_element_type=jnp.float32)
        + b3_ref[...])
    t = jnp.dot(upd, w4_ref[...],
                preferred_element_type=jnp.float32) + b4_ref[...] + h
    hn = _bn(t, gm_ref[...], bt_ref[...])
    h_ref_o[...] = hn
    a_ref_o[...] = jnp.dot(hn, w1a_ref[...],
                           preferred_element_type=jnp.float32) + b1_ref[...]
    b_ref_o[...] = jnp.dot(hn, w1b_ref[...], preferred_element_type=jnp.float32)


def _node_update(h, p0, p1, w3a, w3b, b3, w4, b4, gm, bt, w1a, b1, w1b):
    return pl.pallas_call(
        _update_body,
        out_shape=[jax.ShapeDtypeStruct((N, D), jnp.float32)] * 3,
    )(h, p0, p1, w3a, w3b, b3, w4, b4, gm, bt, w1a, b1, w1b)


def _final_body(h_ref, p0_ref, p1_ref, w3a_ref, w3b_ref, b3_ref, w4_ref,
                b4_ref, gm_ref, bt_ref, ow1_ref, ob1_ref, ow2_ref, ob2_ref,
                o_ref):
    h = h_ref[...]
    aggr = (p0_ref[0] + p0_ref[1]) + (p1_ref[0] + p1_ref[1])
    upd = jax.nn.softplus(
        jnp.dot(h, w3a_ref[...], preferred_element_type=jnp.float32)
        + jnp.dot(aggr, w3b_ref[...], preferred_element_type=jnp.float32)
        + b3_ref[...])
    t = jnp.dot(upd, w4_ref[...],
                preferred_element_type=jnp.float32) + b4_ref[...] + h
    hn = _bn(t, gm_ref[...], bt_ref[...])
    pooled = jnp.mean(hn, axis=0, keepdims=True)
    o1 = jax.nn.softplus(
        jnp.dot(pooled, ow1_ref[...], preferred_element_type=jnp.float32)
        + ob1_ref[...])
    o_ref[...] = jnp.dot(o1, ow2_ref[...],
                         preferred_element_type=jnp.float32) + ob2_ref[...]


def _node_final(h, p0, p1, w3a, w3b, b3, w4, b4, gm, bt, ow1, ob1, ow2, ob2):
    return pl.pallas_call(
        _final_body,
        out_shape=jax.ShapeDtypeStruct((1, 1), jnp.float32),
    )(h, p0, p1, w3a, w3b, b3, w4, b4, gm, bt, ow1, ob1, ow2, ob2)


def kernel(x, edge_attr, params, edge_index, batch):
    del batch
    src = edge_index[0]
    dst = edge_index[1]
    pad_e = EPAD - E
    dstp = jnp.concatenate([dst, jnp.zeros((pad_e,), jnp.int32)])
    srcp = jnp.concatenate([src, jnp.zeros((pad_e,), jnp.int32)])
    dst2 = dstp.reshape(EPAD // UNIT, UNIT)
    src2 = srcp.reshape(EPAD // UNIT, UNIT)
    zeros_sc = jnp.zeros((N, D), jnp.float32)
    eap = jnp.pad(edge_attr, ((0, pad_e), (0, 0)))

    xp = jnp.pad(x, ((0, 0), (0, 3)))
    npw = jnp.pad(params["np_W"], ((0, 3), (0, 0)))
    row = lambda v: v.reshape(1, -1)

    w1 = params["conv_W1"]
    w1a = [w1[l, :D] for l in range(NCONV)]
    w1b = [w1[l, D:2 * D] for l in range(NCONV)]
    w1c = [w1[l, 2 * D:] for l in range(NCONV)]
    b1 = [row(params["conv_b1"][l]) for l in range(NCONV)]
    w2 = [params["conv_W2"][l] for l in range(NCONV)]
    b2 = [row(params["conv_b2"][l]) for l in range(NCONV)]
    w3 = params["conv_W3"]
    w3a = [w3[l, :D] for l in range(NCONV)]
    w3b = [w3[l, D:] for l in range(NCONV)]
    b3 = [row(params["conv_b3"][l]) for l in range(NCONV)]
    w4 = [params["conv_W4"][l] for l in range(NCONV)]
    b4 = [row(params["conv_b4"][l]) for l in range(NCONV)]
    gm = [row(params["bn_gamma"][l]) for l in range(NCONV)]
    bt = [row(params["bn_beta"][l]) for l in range(NCONV)]

    h, a, b = _node_init(xp, npw, row(params["np_b"]),
                         row(params["np_gamma"]), row(params["np_beta"]),
                         w1a[0], b1[0], w1b[0])

    dst_c = [dst2[c * CU:(c + 1) * CU] for c in range(NCHUNK)]
    src_c = [src2[c * CU:(c + 1) * CU] for c in range(NCHUNK)]
    eap_c = [eap[c * CE:(c + 1) * CE] for c in range(NCHUNK)]

    for l in range(NCONV):
        ps = []
        for c in range(NCHUNK):
            g = _sc_gather(a, b, dst_c[c], src_c[c])
            m2 = _edge_mlp(c, eap_c[c], g, params["ep_W"],
                           row(params["ep_b"]), w1c[l], w2[l], b2[l])
            ps.append(_sc_scatter(m2, dst_c[c], zeros_sc))
        if l < NCONV - 1:
            h, a, b = _node_update(h, ps[0], ps[1], w3a[l], w3b[l], b3[l],
                                   w4[l], b4[l], gm[l], bt[l],
                                   w1a[l + 1], b1[l + 1], w1b[l + 1])
        else:
            o = _node_final(h, ps[0], ps[1], w3a[l], w3b[l], b3[l], w4[l],
                            b4[l], gm[l], bt[l], params["out_W1"],
                            row(params["out_b1"]), params["out_W2"],
                            row(params["out_b2"]))
    return o
